# Initial kernel scaffold; baseline (speedup 1.0000x reference)
import functools

import jax
import jax.numpy as jnp
from jax import lax
from jax.experimental import pallas as pl
from jax.experimental.pallas import tpu as pltpu
from jax.experimental.pallas import tpu_sc as plsc

_D = 64
_IDXV = 128
_K = 8
_GROUP = _K * _IDXV


def _make_gather(n_idx: int):
    info = plsc.get_sparse_core_info()
    nc, ns = info.num_cores, info.num_subcores
    nw = nc * ns
    assert n_idx % (nw * _GROUP) == 0
    groups = n_idx // (nw * _GROUP)

    mesh = plsc.VectorSubcoreMesh(core_axis_name="c", subcore_axis_name="s")

    @functools.partial(
        pl.kernel,
        mesh=mesh,
        out_type=jax.ShapeDtypeStruct((n_idx, _D), jnp.float32),
        scratch_types=[
            pltpu.VMEM((_K, _IDXV), jnp.int32),
            pltpu.VMEM((_GROUP, _D), jnp.float32),
            pltpu.SemaphoreType.DMA,
        ],
    )
    def gather_kernel(x_hbm, table_hbm, out_hbm, idx_v, rows_v, sem):
        w = lax.axis_index("s") * nc + lax.axis_index("c")

        def group_body(g, carry):
            gidx = w * groups + g
            pltpu.sync_copy(x_hbm.at[pl.ds(gidx * _K, _K)], idx_v)
            for j in range(_K):
                for i in range(_IDXV // 16):
                    s = pl.ds(i * 16, 16)
                    idx_v[j, s] = ~idx_v[j, s]
            cps = [
                pltpu.async_copy(
                    table_hbm.at[idx_v.at[j]],
                    rows_v.at[pl.ds(j * _IDXV, _IDXV)],
                    sem,
                )
                for j in range(_K)
            ]
            for cp in cps:
                cp.wait()
            pltpu.sync_copy(rows_v, out_hbm.at[pl.ds(gidx * _GROUP, _GROUP)])
            return carry

        lax.fori_loop(0, groups, group_body, 0)

    return gather_kernel


def kernel(x, table):
    b, h = x.shape
    n = b * h
    xf = x.reshape(n // _IDXV, _IDXV)
    out = _make_gather(n)(xf, table)
    return out.reshape(b, h, _D)


# speedup vs baseline: 4.1330x; 4.1330x over previous
import functools

import jax
import jax.numpy as jnp
from jax import lax
from jax.experimental import pallas as pl
from jax.experimental.pallas import tpu as pltpu
from jax.experimental.pallas import tpu_sc as plsc

_D = 64
_IDXV = 128
_K = 8
_GROUP = _K * _IDXV


def _make_gather(n_idx: int):
    info = plsc.get_sparse_core_info()
    nc, ns = info.num_cores, info.num_subcores
    nw = nc * ns
    assert n_idx % (nw * _GROUP) == 0
    groups = n_idx // (nw * _GROUP)

    mesh = plsc.VectorSubcoreMesh(core_axis_name="c", subcore_axis_name="s")

    @functools.partial(
        pl.kernel,
        mesh=mesh,
        out_type=jax.ShapeDtypeStruct((n_idx, _D), jnp.float32),
        scratch_types=[
            pltpu.VMEM((_K, _IDXV), jnp.int32),
            pltpu.VMEM((_GROUP, _D), jnp.float32),
            pltpu.SemaphoreType.DMA,
        ],
        compiler_params=pltpu.CompilerParams(use_tc_tiling_on_sc=False),
    )
    def gather_kernel(x_hbm, table_hbm, out_hbm, idx_v, rows_v, sem):
        w = lax.axis_index("s") * nc + lax.axis_index("c")

        def group_body(g, carry):
            gidx = w * groups + g
            pltpu.sync_copy(x_hbm.at[pl.ds(gidx * _K, _K)], idx_v)
            for j in range(_K):
                for i in range(_IDXV // 16):
                    s = pl.ds(i * 16, 16)
                    idx_v[j, s] = ~idx_v[j, s]
            cps = [
                pltpu.async_copy(
                    table_hbm.at[idx_v.at[j]],
                    rows_v.at[pl.ds(j * _IDXV, _IDXV)],
                    sem,
                )
                for j in range(_K)
            ]
            for cp in cps:
                cp.wait()
            pltpu.sync_copy(rows_v, out_hbm.at[pl.ds(gidx * _GROUP, _GROUP)])
            return carry

        lax.fori_loop(0, groups, group_body, 0)

    return gather_kernel


def kernel(x, table):
    b, h = x.shape
    n = b * h
    xf = x.reshape(n // _IDXV, _IDXV)
    out = _make_gather(n)(xf, table)
    return out.reshape(b, h, _D)


# speedup vs baseline: 4.2409x; 1.0261x over previous
import functools

import jax
import jax.numpy as jnp
from jax import lax
from jax.experimental import pallas as pl
from jax.experimental.pallas import tpu as pltpu
from jax.experimental.pallas import tpu_sc as plsc

_D = 64
_IDXV = 128
_K = 4
_GROUP = _K * _IDXV


def _make_gather(n_idx: int):
    info = plsc.get_sparse_core_info()
    nc, ns = info.num_cores, info.num_subcores
    nw = nc * ns
    assert n_idx % (nw * 2 * _GROUP) == 0
    groups = n_idx // (nw * _GROUP)
    pairs = groups // 2

    mesh = plsc.VectorSubcoreMesh(core_axis_name="c", subcore_axis_name="s")

    @functools.partial(
        pl.kernel,
        mesh=mesh,
        out_type=jax.ShapeDtypeStruct((n_idx, _D), jnp.float32),
        scratch_types=[
            pltpu.VMEM((_K, _IDXV), jnp.int32),
            pltpu.VMEM((_K, _IDXV), jnp.int32),
            pltpu.VMEM((_GROUP, _D), jnp.float32),
            pltpu.VMEM((_GROUP, _D), jnp.float32),
            pltpu.SemaphoreType.DMA,
            pltpu.SemaphoreType.DMA,
            pltpu.SemaphoreType.DMA,
        ],
        compiler_params=pltpu.CompilerParams(use_tc_tiling_on_sc=False),
    )
    def gather_kernel(x_hbm, table_hbm, out_hbm, idx0, idx1, rows0, rows1,
                      sem_g, sem_w0, sem_w1):
        w = lax.axis_index("s") * nc + lax.axis_index("c")
        g_base = w * groups

        def stage(g, idx_v):
            pltpu.sync_copy(x_hbm.at[pl.ds((g_base + g) * _K, _K)], idx_v)
            for j in range(_K):
                for i in range(_IDXV // 16):
                    s = pl.ds(i * 16, 16)
                    idx_v[j, s] = ~idx_v[j, s]

        def half(t, g, idx_v, rows_v, sem_w, stage_next):
            @pl.when(t > 0)
            def _():
                pltpu.make_async_copy(
                    rows_v, out_hbm.at[pl.ds(0, _GROUP)], sem_w).wait()
            cps = [
                pltpu.async_copy(
                    table_hbm.at[idx_v.at[j]],
                    rows_v.at[pl.ds(j * _IDXV, _IDXV)],
                    sem_g,
                )
                for j in range(_K)
            ]
            stage_next()
            for cp in cps:
                cp.wait()
            pltpu.async_copy(
                rows_v, out_hbm.at[pl.ds((g_base + g) * _GROUP, _GROUP)],
                sem_w)

        stage(0, idx0)

        def pair_body(t, carry):
            g0 = 2 * t
            half(t, g0, idx0, rows0, sem_w0, lambda: stage(g0 + 1, idx1))

            def stage_next_even():
                @pl.when(t + 1 < pairs)
                def _():
                    stage(g0 + 2, idx0)

            half(t, g0 + 1, idx1, rows1, sem_w1, stage_next_even)
            return carry

        lax.fori_loop(0, pairs, pair_body, 0)
        pltpu.make_async_copy(rows0, out_hbm.at[pl.ds(0, _GROUP)], sem_w0).wait()
        pltpu.make_async_copy(rows1, out_hbm.at[pl.ds(0, _GROUP)], sem_w1).wait()

    return gather_kernel


def kernel(x, table):
    b, h = x.shape
    n = b * h
    xf = x.reshape(n // _IDXV, _IDXV)
    out = _make_gather(n)(xf, table)
    return out.reshape(b, h, _D)
